# baseline (speedup 1.0000x reference)
import functools

import jax
import jax.numpy as jnp
from jax import lax
from jax.experimental import pallas as pl
from jax.experimental.pallas import tpu as pltpu
from jax.experimental.pallas import tpu_sc as plsc

NN = 10000
NE = 320000
NC = 2
NS = 16
NW = NC * NS
CHUNK = 256
NCHUNKS = 1280
NE_PAD = NCHUNKS * CHUNK
KPW = NCHUNKS // NW
ACC_ROWS = 10240
RPS = ACC_ROWS // NS
W1A = 64
W2A = 16
DEGC = 512
DKPW = NE_PAD // (NW * DEGC)

_mesh = plsc.VectorSubcoreMesh(core_axis_name="c", subcore_axis_name="s")


def _make_sc_segsum(width):
    @functools.partial(
        pl.kernel,
        mesh=_mesh,
        compiler_params=pltpu.CompilerParams(use_tc_tiling_on_sc=False),
        out_type=jax.ShapeDtypeStruct((NC, ACC_ROWS, width), jnp.float32),
        scratch_types=[
            pltpu.VMEM((KPW, CHUNK), jnp.int32),
            pltpu.VMEM((KPW, CHUNK), jnp.int32),
            pltpu.VMEM((CHUNK, width), jnp.float32),
            pltpu.VMEM((CHUNK, width), jnp.float32),
            pltpu.VMEM_SHARED((ACC_ROWS, width), jnp.float32),
            pltpu.SemaphoreType.DMA,
            pltpu.SemaphoreType.DMA,
        ],
    )
    def segsum(table_hbm, src_hbm, dst_hbm, zero_hbm, out_hbm,
               idx_s, idx_d, rows0, rows1, acc, gsem0, gsem1):
        c = lax.axis_index("c")
        s = lax.axis_index("s")
        wid = c * NS + s
        base = s * RPS
        pltpu.sync_copy(zero_hbm.at[pl.ds(base, RPS)], acc.at[pl.ds(base, RPS)])
        pltpu.sync_copy(src_hbm.at[pl.ds(wid * KPW, KPW)], idx_s)
        pltpu.sync_copy(dst_hbm.at[pl.ds(wid * KPW, KPW)], idx_d)
        plsc.subcore_barrier()

        pltpu.async_copy(table_hbm.at[idx_s.at[0]], rows0, gsem0)

        @pl.loop(0, KPW, step=2)
        def _(j):
            pltpu.async_copy(table_hbm.at[idx_s.at[j + 1]], rows1, gsem1)
            pltpu.make_async_copy(table_hbm.at[idx_s.at[j]], rows0, gsem0
                                  ).wait()
            pltpu.sync_copy(rows0, acc.at[idx_d.at[j]], add=True)

            @pl.when(j + 2 < KPW)
            def _():
                pltpu.async_copy(table_hbm.at[idx_s.at[j + 2]], rows0, gsem0)

            pltpu.make_async_copy(table_hbm.at[idx_s.at[j + 1]], rows1,
                                  gsem1).wait()
            pltpu.sync_copy(rows1, acc.at[idx_d.at[j + 1]], add=True)

        plsc.subcore_barrier()
        pltpu.sync_copy(acc.at[pl.ds(base, RPS)],
                        out_hbm.at[c].at[pl.ds(base, RPS)])

    return segsum


_sc_segsum_l1 = _make_sc_segsum(W1A)
_sc_segsum_l2 = _make_sc_segsum(W2A)


@functools.partial(
    pl.kernel,
    mesh=_mesh,
    compiler_params=pltpu.CompilerParams(use_tc_tiling_on_sc=False),
    out_type=jax.ShapeDtypeStruct((NC, ACC_ROWS, W2A), jnp.float32),
    scratch_types=[
        pltpu.VMEM((DKPW, DEGC), jnp.int32),
        pltpu.VMEM((DEGC, W2A), jnp.float32),
        pltpu.VMEM_SHARED((ACC_ROWS, W2A), jnp.float32),
    ],
)
def _sc_degree(dst_hbm, ones_hbm, zero_hbm, out_hbm, idx_d, ones, acc):
    c = lax.axis_index("c")
    s = lax.axis_index("s")
    wid = c * NS + s
    base = s * RPS
    pltpu.sync_copy(zero_hbm.at[pl.ds(base, RPS)], acc.at[pl.ds(base, RPS)])
    pltpu.sync_copy(ones_hbm, ones)
    pltpu.sync_copy(dst_hbm.at[wid], idx_d)
    plsc.subcore_barrier()

    @pl.loop(0, DKPW)
    def _(j):
        pltpu.sync_copy(ones, acc.at[idx_d.at[j]], add=True)

    plsc.subcore_barrier()
    pltpu.sync_copy(acc.at[pl.ds(base, RPS)],
                    out_hbm.at[c].at[pl.ds(base, RPS)])


def _tc1_body(x_ref, wl_ref, wr_ref, y1_ref, xr_ref):
    x = x_ref[...]
    y1_ref[...] = jnp.dot(x, wl_ref[...], preferred_element_type=jnp.float32)
    xr_ref[...] = jnp.dot(x, wr_ref[...], preferred_element_type=jnp.float32)


def _tc2_body(acc_ref, deg_ref, xr_ref, b1_ref, w2l_ref, w2r_ref,
              y2_ref, hr_ref, inv_ref):
    feat = acc_ref[0, :NN, :] + acc_ref[1, :NN, :]
    deg = deg_ref[0, :NN, :1] + deg_ref[1, :NN, :1]
    inv = 1.0 / jnp.maximum(deg, 1.0)
    h = jnp.maximum(feat * inv + b1_ref[...] + xr_ref[...], 0.0)
    y2_ref[...] = jnp.dot(h, w2l_ref[...], preferred_element_type=jnp.float32)
    hr_ref[...] = jnp.dot(h, w2r_ref[...], preferred_element_type=jnp.float32)
    inv_ref[...] = jnp.broadcast_to(inv, (NN, W2A))


def _tc3_body(acc_ref, inv_ref, hr_ref, b2_ref, out_ref):
    ssum = acc_ref[0, :NN, :] + acc_ref[1, :NN, :]
    out_ref[...] = ssum * inv_ref[...] + hr_ref[...] + b2_ref[...]


def kernel(x, edge_index, W1l, b1, W1r, W2l, b2, W2r):
    src = edge_index[0].astype(jnp.int32)
    dst = edge_index[1].astype(jnp.int32)
    pad = NE_PAD - NE
    srcp = jnp.concatenate([src, jnp.zeros((pad,), jnp.int32)]
                           ).reshape(NCHUNKS, CHUNK)
    trash = NN + jnp.arange(pad, dtype=jnp.int32) % (ACC_ROWS - NN)
    dstp = jnp.concatenate([dst, trash]).reshape(NCHUNKS, CHUNK)
    dstf = dstp.reshape(NW, DKPW, DEGC)
    z1 = jnp.zeros((ACC_ROWS, W1A), jnp.float32)
    z2 = jnp.zeros((ACC_ROWS, W2A), jnp.float32)
    onehot = jnp.zeros((DEGC, W2A), jnp.float32).at[:, 0].set(1.0)
    w2l_p = jnp.pad(W2l, ((0, 0), (0, W2A - 2)))
    w2r_p = jnp.pad(W2r, ((0, 0), (0, W2A - 2)))
    b1r = jnp.reshape(b1, (1, 64))
    b2r = jnp.reshape(jnp.pad(b2, (0, W2A - 2)), (1, W2A))

    degacc = _sc_degree(dstf, onehot, z2)

    y1, xr = pl.pallas_call(
        _tc1_body,
        out_shape=[jax.ShapeDtypeStruct((NN, W1A), jnp.float32),
                   jax.ShapeDtypeStruct((NN, 64), jnp.float32)],
    )(x, W1l, W1r)

    acc1 = _sc_segsum_l1(y1, srcp, dstp, z1)

    y2p, hr, inv = pl.pallas_call(
        _tc2_body,
        out_shape=[jax.ShapeDtypeStruct((NN, W2A), jnp.float32),
                   jax.ShapeDtypeStruct((NN, W2A), jnp.float32),
                   jax.ShapeDtypeStruct((NN, W2A), jnp.float32)],
    )(acc1, degacc, xr, b1r, w2l_p, w2r_p)

    acc2 = _sc_segsum_l2(y2p, srcp, dstp, z2)

    out16 = pl.pallas_call(
        _tc3_body,
        out_shape=jax.ShapeDtypeStruct((NN, W2A), jnp.float32),
    )(acc2, inv, hr, b2r)

    return out16[:, :2]


# speedup vs baseline: 1.8967x; 1.8967x over previous
import functools

import jax
import jax.numpy as jnp
from jax import lax
from jax.experimental import pallas as pl
from jax.experimental.pallas import tpu as pltpu
from jax.experimental.pallas import tpu_sc as plsc

NN = 10000
NE = 320000
NC = 2
NS = 16
NW = NC * NS
CHUNK = 192
NCHUNKS = 1728
NE_PAD = NCHUNKS * CHUNK
KPW = NCHUNKS // NW
ACC_ROWS = 10240
RPS = ACC_ROWS // NS
W1A = 64
W2A = 16
DEGC = 576
DKPW = NE_PAD // (NW * DEGC)

_mesh = plsc.VectorSubcoreMesh(core_axis_name="c", subcore_axis_name="s")


def _make_sc_segsum(width):
    @functools.partial(
        pl.kernel,
        mesh=_mesh,
        compiler_params=pltpu.CompilerParams(use_tc_tiling_on_sc=False),
        out_type=jax.ShapeDtypeStruct((NC, ACC_ROWS, width), jnp.float32),
        scratch_types=[
            pltpu.VMEM((KPW, CHUNK), jnp.int32),
            pltpu.VMEM((KPW, CHUNK), jnp.int32),
            pltpu.VMEM((CHUNK, width), jnp.float32),
            pltpu.VMEM((CHUNK, width), jnp.float32),
            pltpu.VMEM_SHARED((ACC_ROWS, width), jnp.float32),
            pltpu.VMEM_SHARED((NN, width), jnp.float32),
            pltpu.SemaphoreType.DMA,
            pltpu.SemaphoreType.DMA,
        ],
    )
    def segsum(table_hbm, src_hbm, dst_hbm, zero_hbm, out_hbm,
               idx_s, idx_d, rows0, rows1, acc, tab, gsem0, gsem1):
        c = lax.axis_index("c")
        s = lax.axis_index("s")
        wid = c * NS + s
        base = s * RPS
        pltpu.sync_copy(zero_hbm.at[pl.ds(base, RPS)], acc.at[pl.ds(base, RPS)])
        pltpu.sync_copy(table_hbm.at[pl.ds(s * (NN // NS), NN // NS)],
                        tab.at[pl.ds(s * (NN // NS), NN // NS)])
        pltpu.sync_copy(src_hbm.at[pl.ds(wid * KPW, KPW)], idx_s)
        pltpu.sync_copy(dst_hbm.at[pl.ds(wid * KPW, KPW)], idx_d)
        plsc.subcore_barrier()

        pltpu.async_copy(tab.at[idx_s.at[0]], rows0, gsem0)

        @pl.loop(0, KPW, step=2)
        def _(j):
            pltpu.async_copy(tab.at[idx_s.at[j + 1]], rows1, gsem1)
            pltpu.make_async_copy(tab.at[idx_s.at[j]], rows0, gsem0
                                  ).wait()
            pltpu.sync_copy(rows0, acc.at[idx_d.at[j]], add=True)

            @pl.when(j + 2 < KPW)
            def _():
                pltpu.async_copy(tab.at[idx_s.at[j + 2]], rows0, gsem0)

            pltpu.make_async_copy(tab.at[idx_s.at[j + 1]], rows1,
                                  gsem1).wait()
            pltpu.sync_copy(rows1, acc.at[idx_d.at[j + 1]], add=True)

        plsc.subcore_barrier()
        pltpu.sync_copy(acc.at[pl.ds(base, RPS)],
                        out_hbm.at[c].at[pl.ds(base, RPS)])

    return segsum


_sc_segsum_l1 = _make_sc_segsum(W1A)
_sc_segsum_l2 = _make_sc_segsum(W2A)


@functools.partial(
    pl.kernel,
    mesh=_mesh,
    compiler_params=pltpu.CompilerParams(use_tc_tiling_on_sc=False),
    out_type=jax.ShapeDtypeStruct((NC, ACC_ROWS, W2A), jnp.float32),
    scratch_types=[
        pltpu.VMEM((DKPW, DEGC), jnp.int32),
        pltpu.VMEM((DEGC, W2A), jnp.float32),
        pltpu.VMEM_SHARED((ACC_ROWS, W2A), jnp.float32),
    ],
)
def _sc_degree(dst_hbm, ones_hbm, zero_hbm, out_hbm, idx_d, ones, acc):
    c = lax.axis_index("c")
    s = lax.axis_index("s")
    wid = c * NS + s
    base = s * RPS
    pltpu.sync_copy(zero_hbm.at[pl.ds(base, RPS)], acc.at[pl.ds(base, RPS)])
    pltpu.sync_copy(ones_hbm, ones)
    pltpu.sync_copy(dst_hbm.at[wid], idx_d)
    plsc.subcore_barrier()

    @pl.loop(0, DKPW)
    def _(j):
        pltpu.sync_copy(ones, acc.at[idx_d.at[j]], add=True)

    plsc.subcore_barrier()
    pltpu.sync_copy(acc.at[pl.ds(base, RPS)],
                    out_hbm.at[c].at[pl.ds(base, RPS)])


def _tc1_body(x_ref, wl_ref, wr_ref, y1_ref, xr_ref):
    x = x_ref[...]
    y1_ref[...] = jnp.dot(x, wl_ref[...], preferred_element_type=jnp.float32)
    xr_ref[...] = jnp.dot(x, wr_ref[...], preferred_element_type=jnp.float32)


def _tc2_body(acc_ref, deg_ref, xr_ref, b1_ref, w2l_ref, w2r_ref,
              y2_ref, hr_ref, inv_ref):
    feat = acc_ref[0, :NN, :] + acc_ref[1, :NN, :]
    deg = deg_ref[0, :NN, :1] + deg_ref[1, :NN, :1]
    inv = 1.0 / jnp.maximum(deg, 1.0)
    h = jnp.maximum(feat * inv + b1_ref[...] + xr_ref[...], 0.0)
    y2_ref[...] = jnp.dot(h, w2l_ref[...], preferred_element_type=jnp.float32)
    hr_ref[...] = jnp.dot(h, w2r_ref[...], preferred_element_type=jnp.float32)
    inv_ref[...] = jnp.broadcast_to(inv, (NN, W2A))


def _tc3_body(acc_ref, inv_ref, hr_ref, b2_ref, out_ref):
    ssum = acc_ref[0, :NN, :] + acc_ref[1, :NN, :]
    out_ref[...] = ssum * inv_ref[...] + hr_ref[...] + b2_ref[...]


def kernel(x, edge_index, W1l, b1, W1r, W2l, b2, W2r):
    src = edge_index[0].astype(jnp.int32)
    dst = edge_index[1].astype(jnp.int32)
    pad = NE_PAD - NE
    srcp = jnp.concatenate([src, jnp.zeros((pad,), jnp.int32)]
                           ).reshape(NCHUNKS, CHUNK)
    trash = NN + jnp.arange(pad, dtype=jnp.int32) % (ACC_ROWS - NN)
    dstp = jnp.concatenate([dst, trash]).reshape(NCHUNKS, CHUNK)
    dstf = dstp.reshape(NW, DKPW, DEGC)
    z1 = jnp.zeros((ACC_ROWS, W1A), jnp.float32)
    z2 = jnp.zeros((ACC_ROWS, W2A), jnp.float32)
    onehot = jnp.zeros((DEGC, W2A), jnp.float32).at[:, 0].set(1.0)
    w2l_p = jnp.pad(W2l, ((0, 0), (0, W2A - 2)))
    w2r_p = jnp.pad(W2r, ((0, 0), (0, W2A - 2)))
    b1r = jnp.reshape(b1, (1, 64))
    b2r = jnp.reshape(jnp.pad(b2, (0, W2A - 2)), (1, W2A))

    degacc = _sc_degree(dstf, onehot, z2)

    y1, xr = pl.pallas_call(
        _tc1_body,
        out_shape=[jax.ShapeDtypeStruct((NN, W1A), jnp.float32),
                   jax.ShapeDtypeStruct((NN, 64), jnp.float32)],
    )(x, W1l, W1r)

    acc1 = _sc_segsum_l1(y1, srcp, dstp, z1)

    y2p, hr, inv = pl.pallas_call(
        _tc2_body,
        out_shape=[jax.ShapeDtypeStruct((NN, W2A), jnp.float32),
                   jax.ShapeDtypeStruct((NN, W2A), jnp.float32),
                   jax.ShapeDtypeStruct((NN, W2A), jnp.float32)],
    )(acc1, degacc, xr, b1r, w2l_p, w2r_p)

    acc2 = _sc_segsum_l2(y2p, srcp, dstp, z2)

    out16 = pl.pallas_call(
        _tc3_body,
        out_shape=jax.ShapeDtypeStruct((NN, W2A), jnp.float32),
    )(acc2, inv, hr, b2r)

    return out16[:, :2]
